# Initial kernel scaffold; baseline (speedup 1.0000x reference)
import functools

import jax
import jax.numpy as jnp
from jax import lax
from jax.experimental import pallas as pl
from jax.experimental.pallas import tpu as pltpu
from jax.experimental.pallas import tpu_sc as plsc

N = 100000
E = 1600000
D = 32
H = 32
G = 16
B = 128
T = 5

NC = 2
NS = 16
HH = H // 2

NP = 100352
TRASH = N
EP = 1605632
CH = 128
KJ = 8
NROW = EP // (16 * CH)
NMACRO = NROW // KJ
DROW = EP // (2 * 16 * CH)
DMACRO = DROW // KJ
SL = NP // 16

_MESH = plsc.VectorSubcoreMesh(
    core_axis_name="c", subcore_axis_name="s", num_cores=NC, num_subcores=NS)


@functools.partial(
    pl.kernel,
    out_type=jax.ShapeDtypeStruct((NC, NP), jnp.float32),
    mesh=_MESH,
    scratch_types=[
        pltpu.VMEM_SHARED((NP,), jnp.float32),
        pltpu.VMEM((KJ, CH), jnp.int32),
        pltpu.VMEM((CH,), jnp.float32),
        pltpu.SemaphoreType.DMA,
    ],
)
def _sc_degree(dst_hbm, zeros_hbm, out_hbm, acc, dbuf, onesv, ssem):
    c = lax.axis_index("c")
    t = lax.axis_index("s")
    pltpu.sync_copy(zeros_hbm.at[pl.ds(t * SL, SL)], acc.at[pl.ds(t * SL, SL)])
    for i in range(CH // 16):
        onesv[pl.ds(i * 16, 16)] = jnp.full((16,), 1.0, jnp.float32)
    plsc.subcore_barrier()

    def macro(kb, carry):
        pltpu.sync_copy(dst_hbm.at[c, t, pl.ds(kb * KJ, KJ)], dbuf)
        descs = [
            pltpu.async_copy(onesv, acc.at[dbuf.at[j]], ssem, add=True)
            for j in range(KJ)
        ]
        for d in descs:
            d.wait()
        return carry

    lax.fori_loop(0, DMACRO, macro, 0)
    plsc.subcore_barrier()
    pltpu.sync_copy(acc.at[pl.ds(t * SL, SL)], out_hbm.at[c, pl.ds(t * SL, SL)])


@functools.partial(
    pl.kernel,
    out_type=jax.ShapeDtypeStruct((NC, NP, HH), jnp.float32),
    mesh=_MESH,
    scratch_types=[
        pltpu.VMEM_SHARED((NP, HH), jnp.float32),
        pltpu.VMEM((KJ, CH), jnp.int32),
        pltpu.VMEM((KJ, CH), jnp.int32),
        pltpu.VMEM((KJ, CH, HH), jnp.float32),
        pltpu.SemaphoreType.DMA,
        pltpu.SemaphoreType.DMA,
    ],
)
def _sc_aggregate(s_hbm, src_hbm, dst_hbm, out_hbm,
                  acc, sbuf, dbuf, rows, gsem, ssem):
    c = lax.axis_index("c")
    t = lax.axis_index("s")
    pltpu.sync_copy(s_hbm.at[c, pl.ds(t * SL, SL)], acc.at[pl.ds(t * SL, SL)])
    plsc.subcore_barrier()

    def macro(kb, carry):
        pltpu.sync_copy(src_hbm.at[t, pl.ds(kb * KJ, KJ)], sbuf)
        pltpu.sync_copy(dst_hbm.at[t, pl.ds(kb * KJ, KJ)], dbuf)
        gd = [
            pltpu.async_copy(s_hbm.at[c].at[sbuf.at[j]], rows.at[j], gsem)
            for j in range(KJ)
        ]
        for d in gd:
            d.wait()
        sd = [
            pltpu.async_copy(rows.at[j], acc.at[dbuf.at[j]], ssem, add=True)
            for j in range(KJ)
        ]
        for d in sd:
            d.wait()
        return carry

    lax.fori_loop(0, NMACRO, macro, 0)
    plsc.subcore_barrier()
    pltpu.sync_copy(acc.at[pl.ds(t * SL, SL)], out_hbm.at[c, pl.ds(t * SL, SL)])


BN_R = 1024
NG = NP // BN_R


def _prep_body(p_ref, x_ref, dq_ref, s_ref):
    total = p_ref[0] + p_ref[1]
    dq = lax.rsqrt(1.0 + total)
    dq_ref[...] = dq
    s = x_ref[...] * dq
    s_ref[0] = s[:, :HH]
    s_ref[1] = s[:, HH:]


def _tc_prep(p, x):
    return pl.pallas_call(
        _prep_body,
        grid=(NG,),
        in_specs=[
            pl.BlockSpec((NC, BN_R, 1), lambda i: (0, i, 0)),
            pl.BlockSpec((BN_R, D), lambda i: (i, 0)),
        ],
        out_specs=[
            pl.BlockSpec((BN_R, 1), lambda i: (i, 0)),
            pl.BlockSpec((NC, BN_R, HH), lambda i: (0, i, 0)),
        ],
        out_shape=[
            jax.ShapeDtypeStruct((NP, 1), jnp.float32),
            jax.ShapeDtypeStruct((NC, NP, HH), jnp.float32),
        ],
    )(p.reshape(NC, NP, 1), x)


def _matstats_body(a_ref, dq_ref, wt_ref, wb_ref, b_ref,
                   y_ref, ssum_ref, ssq_ref):
    i = pl.program_id(0)
    dq = dq_ref[...]
    y = (jnp.dot(a_ref[0] * dq, wt_ref[...], preferred_element_type=jnp.float32)
         + jnp.dot(a_ref[1] * dq, wb_ref[...], preferred_element_type=jnp.float32)
         + b_ref[...])
    y_ref[...] = y
    rows = i * BN_R + lax.broadcasted_iota(jnp.int32, (BN_R, 1), 0)
    ym = jnp.where(rows < N, y, 0.0)

    @pl.when(i == 0)
    def _():
        ssum_ref[...] = jnp.zeros_like(ssum_ref)
        ssq_ref[...] = jnp.zeros_like(ssq_ref)

    ssum_ref[...] += jnp.sum(ym, axis=0, keepdims=True)
    ssq_ref[...] += jnp.sum(ym * ym, axis=0, keepdims=True)


def _tc_matstats(agg, dq, wt, wb, b):
    return pl.pallas_call(
        _matstats_body,
        grid=(NG,),
        in_specs=[
            pl.BlockSpec((NC, BN_R, HH), lambda i: (0, i, 0)),
            pl.BlockSpec((BN_R, 1), lambda i: (i, 0)),
            pl.BlockSpec((HH, H), lambda i: (0, 0)),
            pl.BlockSpec((HH, H), lambda i: (0, 0)),
            pl.BlockSpec((1, H), lambda i: (0, 0)),
        ],
        out_specs=[
            pl.BlockSpec((BN_R, H), lambda i: (i, 0)),
            pl.BlockSpec((1, H), lambda i: (0, 0)),
            pl.BlockSpec((1, H), lambda i: (0, 0)),
        ],
        out_shape=[
            jax.ShapeDtypeStruct((NP, H), jnp.float32),
            jax.ShapeDtypeStruct((1, H), jnp.float32),
            jax.ShapeDtypeStruct((1, H), jnp.float32),
        ],
        compiler_params=pltpu.CompilerParams(
            dimension_semantics=("arbitrary",)),
    )(agg, dq, wt, wb, b)


def _bnrelu_body(y_ref, ssum_ref, ssq_ref, dq_ref, g_ref, be_ref, s_ref):
    mean = ssum_ref[...] / N
    var = ssq_ref[...] / N - mean * mean
    inv = lax.rsqrt(var + 1e-5)
    h = jnp.maximum((y_ref[...] - mean) * inv * g_ref[...] + be_ref[...], 0.0)
    s = h * dq_ref[...]
    s_ref[0] = s[:, :HH]
    s_ref[1] = s[:, HH:]


def _tc_bnrelu(y, ssum, ssq, dq, g, be):
    return pl.pallas_call(
        _bnrelu_body,
        grid=(NG,),
        in_specs=[
            pl.BlockSpec((BN_R, H), lambda i: (i, 0)),
            pl.BlockSpec((1, H), lambda i: (0, 0)),
            pl.BlockSpec((1, H), lambda i: (0, 0)),
            pl.BlockSpec((BN_R, 1), lambda i: (i, 0)),
            pl.BlockSpec((1, H), lambda i: (0, 0)),
            pl.BlockSpec((1, H), lambda i: (0, 0)),
        ],
        out_specs=[pl.BlockSpec((NC, BN_R, HH), lambda i: (0, i, 0))],
        out_shape=[jax.ShapeDtypeStruct((NC, NP, HH), jnp.float32)],
    )(y, ssum, ssq, dq, g, be)


def _pool_body(a_ref, dq_ref, wt_ref, wb_ref, b_ref, batch_ref,
               psum_ref, pcnt_ref):
    i = pl.program_id(0)
    dq = dq_ref[...]
    y = (jnp.dot(a_ref[0] * dq, wt_ref[...], preferred_element_type=jnp.float32)
         + jnp.dot(a_ref[1] * dq, wb_ref[...], preferred_element_type=jnp.float32)
         + b_ref[...])
    rows = i * BN_R + lax.broadcasted_iota(jnp.int32, (BN_R, 1), 0)
    seg = lax.broadcasted_iota(jnp.int32, (BN_R, B), 1)
    oh = jnp.where((batch_ref[...] == seg) & (rows < N), 1.0, 0.0)

    @pl.when(i == 0)
    def _():
        psum_ref[...] = jnp.zeros_like(psum_ref)
        pcnt_ref[...] = jnp.zeros_like(pcnt_ref)

    psum_ref[...] += lax.dot_general(
        oh, y, (((0,), (0,)), ((), ())), preferred_element_type=jnp.float32)
    pcnt_ref[...] += lax.dot_general(
        oh, jnp.ones((BN_R, 1), jnp.float32), (((0,), (0,)), ((), ())),
        preferred_element_type=jnp.float32)


def _tc_pool(agg, dq, wt, wb, b, batch):
    return pl.pallas_call(
        _pool_body,
        grid=(NG,),
        in_specs=[
            pl.BlockSpec((NC, BN_R, HH), lambda i: (0, i, 0)),
            pl.BlockSpec((BN_R, 1), lambda i: (i, 0)),
            pl.BlockSpec((HH, H), lambda i: (0, 0)),
            pl.BlockSpec((HH, H), lambda i: (0, 0)),
            pl.BlockSpec((1, H), lambda i: (0, 0)),
            pl.BlockSpec((BN_R, 1), lambda i: (i, 0)),
        ],
        out_specs=[
            pl.BlockSpec((B, H), lambda i: (0, 0)),
            pl.BlockSpec((B, 1), lambda i: (0, 0)),
        ],
        out_shape=[
            jax.ShapeDtypeStruct((B, H), jnp.float32),
            jax.ShapeDtypeStruct((B, 1), jnp.float32),
        ],
        compiler_params=pltpu.CompilerParams(
            dimension_semantics=("arbitrary",)),
    )(agg, dq, wt, wb, b, batch)


def _head_body(psum_ref, pcnt_ref, gf_ref, gw_ref, gb_ref,
               w1a_ref, w1b_ref, b1_ref, w2_ref, b2_ref, out_ref):
    pooled = psum_ref[...] / jnp.maximum(pcnt_ref[...], 1.0)
    grepr = jnp.maximum(
        jnp.dot(gf_ref[...], gw_ref[...], preferred_element_type=jnp.float32)
        + gb_ref[...], 0.0)
    hid = jnp.maximum(
        jnp.dot(pooled, w1a_ref[...], preferred_element_type=jnp.float32)
        + jnp.dot(grepr, w1b_ref[...], preferred_element_type=jnp.float32)
        + b1_ref[...], 0.0)
    out_ref[...] = (
        jnp.dot(hid, w2_ref[...], preferred_element_type=jnp.float32)
        + b2_ref[...])


def _tc_head(psum, pcnt, gf, gw, gb, w1a, w1b, b1, w2p, b2p):
    return pl.pallas_call(
        _head_body,
        out_shape=jax.ShapeDtypeStruct((B, 8), jnp.float32),
    )(psum, pcnt, gf, gw, gb, w1a, w1b, b1, w2p, b2p)


def kernel(x, edge_index, batch, global_features, W1, b1, W2, b2, W3, b3,
           g1, be1, g2, be2, gW, gb, pW1, pb1, pW2, pb2):
    f32 = jnp.float32
    pad_e = EP - E
    src_p = jnp.concatenate(
        [edge_index[0], jnp.full((pad_e,), TRASH, jnp.int32)]
    ).reshape(16, NROW, CH)
    dst_flat = jnp.concatenate(
        [edge_index[1], jnp.full((pad_e,), TRASH, jnp.int32)])
    dst_p = dst_flat.reshape(16, NROW, CH)
    dst_deg = dst_flat.reshape(NC, 16, DROW, CH)

    x_p = jnp.concatenate([x, jnp.zeros((NP - N, D), f32)], axis=0)
    batch_p = jnp.concatenate(
        [batch, jnp.zeros((NP - N,), jnp.int32)]).reshape(NP, 1)
    zeros_np = jnp.zeros((NP,), f32)

    w1t, w1b_ = W1[:HH], W1[HH:]
    w2t, w2b_ = W2[:HH], W2[HH:]
    w3t, w3b_ = W3[:HH], W3[HH:]
    b1r, b2r, b3r = b1.reshape(1, H), b2.reshape(1, H), b3.reshape(1, H)
    g1r, be1r = g1.reshape(1, H), be1.reshape(1, H)
    g2r, be2r = g2.reshape(1, H), be2.reshape(1, H)
    gbr = gb.reshape(1, G)
    pW1a, pW1b = pW1[:H], pW1[H:]
    pb1r = pb1.reshape(1, H)
    pW2p = jnp.concatenate([pW2, jnp.zeros((H, 8 - T), f32)], axis=1)
    pb2p = jnp.concatenate([pb2, jnp.zeros((8 - T,), f32)]).reshape(1, 8)

    p = _sc_degree(dst_deg, zeros_np)
    dq, s = _tc_prep(p, x_p)

    agg = _sc_aggregate(s, src_p, dst_p)
    y, ssum, ssq = _tc_matstats(agg, dq, w1t, w1b_, b1r)
    s = _tc_bnrelu(y, ssum, ssq, dq, g1r, be1r)[0]

    agg = _sc_aggregate(s, src_p, dst_p)
    y, ssum, ssq = _tc_matstats(agg, dq, w2t, w2b_, b2r)
    s = _tc_bnrelu(y, ssum, ssq, dq, g2r, be2r)[0]

    agg = _sc_aggregate(s, src_p, dst_p)
    psum, pcnt = _tc_pool(agg, dq, w3t, w3b_, b3r, batch_p)

    out = _tc_head(psum, pcnt, global_features, gW, gbr,
                   pW1a, pW1b, pb1r, pW2p, pb2p)
    return out[:, :T]


# speedup vs baseline: 20.1494x; 20.1494x over previous
import functools

import jax
import jax.numpy as jnp
from jax import lax
from jax.experimental import pallas as pl
from jax.experimental.pallas import tpu as pltpu
from jax.experimental.pallas import tpu_sc as plsc

N = 100000
E = 1600000
D = 32
H = 32
G = 16
B = 128
T = 5

NC = 2
NS = 16
HH = H // 2

NP = 100352
TRASH = N
EP = 1605632
CH = 128
KJ = 8
NROW = EP // (16 * CH)
NMACRO = NROW // KJ
DROW = EP // (2 * 16 * CH)
DMACRO = DROW // KJ
SL = NP // 16

_MESH = plsc.VectorSubcoreMesh(
    core_axis_name="c", subcore_axis_name="s", num_cores=NC, num_subcores=NS)


@functools.partial(
    pl.kernel,
    out_type=jax.ShapeDtypeStruct((NC, NP), jnp.float32),
    mesh=_MESH,
    scratch_types=[
        pltpu.VMEM_SHARED((NP,), jnp.float32),
        pltpu.VMEM((KJ, CH), jnp.int32),
        pltpu.VMEM((CH,), jnp.float32),
        pltpu.SemaphoreType.DMA,
    ],
    compiler_params=pltpu.CompilerParams(use_tc_tiling_on_sc=False),
)
def _sc_degree(dst_hbm, zeros_hbm, out_hbm, acc, dbuf, onesv, ssem):
    c = lax.axis_index("c")
    t = lax.axis_index("s")
    pltpu.sync_copy(zeros_hbm.at[pl.ds(t * SL, SL)], acc.at[pl.ds(t * SL, SL)])
    for i in range(CH // 16):
        onesv[pl.ds(i * 16, 16)] = jnp.full((16,), 1.0, jnp.float32)
    plsc.subcore_barrier()

    def macro(kb, carry):
        pltpu.sync_copy(dst_hbm.at[c, t, pl.ds(kb * KJ, KJ)], dbuf)
        descs = [
            pltpu.async_copy(onesv, acc.at[dbuf.at[j]], ssem, add=True)
            for j in range(KJ)
        ]
        for d in descs:
            d.wait()
        return carry

    lax.fori_loop(0, DMACRO, macro, 0)
    plsc.subcore_barrier()
    pltpu.sync_copy(acc.at[pl.ds(t * SL, SL)], out_hbm.at[c, pl.ds(t * SL, SL)])


@functools.partial(
    pl.kernel,
    out_type=jax.ShapeDtypeStruct((NC, NP, HH), jnp.float32),
    mesh=_MESH,
    scratch_types=[
        pltpu.VMEM_SHARED((NP, HH), jnp.float32),
        pltpu.VMEM((KJ, CH), jnp.int32),
        pltpu.VMEM((KJ, CH), jnp.int32),
        pltpu.VMEM((KJ, CH, HH), jnp.float32),
        pltpu.SemaphoreType.DMA,
        pltpu.SemaphoreType.DMA,
    ],
    compiler_params=pltpu.CompilerParams(use_tc_tiling_on_sc=False),
)
def _sc_aggregate(s_hbm, src_hbm, dst_hbm, out_hbm,
                  acc, sbuf, dbuf, rows, gsem, ssem):
    c = lax.axis_index("c")
    t = lax.axis_index("s")
    pltpu.sync_copy(s_hbm.at[c, pl.ds(t * SL, SL)], acc.at[pl.ds(t * SL, SL)])
    plsc.subcore_barrier()

    def macro(kb, carry):
        pltpu.sync_copy(src_hbm.at[t, pl.ds(kb * KJ, KJ)], sbuf)
        pltpu.sync_copy(dst_hbm.at[t, pl.ds(kb * KJ, KJ)], dbuf)
        gd = [
            pltpu.async_copy(s_hbm.at[c].at[sbuf.at[j]], rows.at[j], gsem)
            for j in range(KJ)
        ]
        for d in gd:
            d.wait()
        sd = [
            pltpu.async_copy(rows.at[j], acc.at[dbuf.at[j]], ssem, add=True)
            for j in range(KJ)
        ]
        for d in sd:
            d.wait()
        return carry

    lax.fori_loop(0, NMACRO, macro, 0)
    plsc.subcore_barrier()
    pltpu.sync_copy(acc.at[pl.ds(t * SL, SL)], out_hbm.at[c, pl.ds(t * SL, SL)])


BN_R = 1024
NG = NP // BN_R


def _prep_body(p_ref, x_ref, dq_ref, s_ref):
    total = p_ref[0] + p_ref[1]
    dq = lax.rsqrt(1.0 + total)
    dq_ref[...] = dq
    s = x_ref[...] * dq
    s_ref[0] = s[:, :HH]
    s_ref[1] = s[:, HH:]


def _tc_prep(p, x):
    return pl.pallas_call(
        _prep_body,
        grid=(NG,),
        in_specs=[
            pl.BlockSpec((NC, BN_R, 1), lambda i: (0, i, 0)),
            pl.BlockSpec((BN_R, D), lambda i: (i, 0)),
        ],
        out_specs=[
            pl.BlockSpec((BN_R, 1), lambda i: (i, 0)),
            pl.BlockSpec((NC, BN_R, HH), lambda i: (0, i, 0)),
        ],
        out_shape=[
            jax.ShapeDtypeStruct((NP, 1), jnp.float32),
            jax.ShapeDtypeStruct((NC, NP, HH), jnp.float32),
        ],
    )(p.reshape(NC, NP, 1), x)


def _matstats_body(a_ref, dq_ref, wt_ref, wb_ref, b_ref,
                   y_ref, ssum_ref, ssq_ref):
    i = pl.program_id(0)
    dq = dq_ref[...]
    y = (jnp.dot(a_ref[0] * dq, wt_ref[...], preferred_element_type=jnp.float32)
         + jnp.dot(a_ref[1] * dq, wb_ref[...], preferred_element_type=jnp.float32)
         + b_ref[...])
    y_ref[...] = y
    rows = i * BN_R + lax.broadcasted_iota(jnp.int32, (BN_R, 1), 0)
    ym = jnp.where(rows < N, y, 0.0)

    @pl.when(i == 0)
    def _():
        ssum_ref[...] = jnp.zeros_like(ssum_ref)
        ssq_ref[...] = jnp.zeros_like(ssq_ref)

    ssum_ref[...] += jnp.sum(ym, axis=0, keepdims=True)
    ssq_ref[...] += jnp.sum(ym * ym, axis=0, keepdims=True)


def _tc_matstats(agg, dq, wt, wb, b):
    return pl.pallas_call(
        _matstats_body,
        grid=(NG,),
        in_specs=[
            pl.BlockSpec((NC, BN_R, HH), lambda i: (0, i, 0)),
            pl.BlockSpec((BN_R, 1), lambda i: (i, 0)),
            pl.BlockSpec((HH, H), lambda i: (0, 0)),
            pl.BlockSpec((HH, H), lambda i: (0, 0)),
            pl.BlockSpec((1, H), lambda i: (0, 0)),
        ],
        out_specs=[
            pl.BlockSpec((BN_R, H), lambda i: (i, 0)),
            pl.BlockSpec((1, H), lambda i: (0, 0)),
            pl.BlockSpec((1, H), lambda i: (0, 0)),
        ],
        out_shape=[
            jax.ShapeDtypeStruct((NP, H), jnp.float32),
            jax.ShapeDtypeStruct((1, H), jnp.float32),
            jax.ShapeDtypeStruct((1, H), jnp.float32),
        ],
        compiler_params=pltpu.CompilerParams(
            dimension_semantics=("arbitrary",)),
    )(agg, dq, wt, wb, b)


def _bnrelu_body(y_ref, ssum_ref, ssq_ref, dq_ref, g_ref, be_ref, s_ref):
    mean = ssum_ref[...] / N
    var = ssq_ref[...] / N - mean * mean
    inv = lax.rsqrt(var + 1e-5)
    h = jnp.maximum((y_ref[...] - mean) * inv * g_ref[...] + be_ref[...], 0.0)
    s = h * dq_ref[...]
    s_ref[0] = s[:, :HH]
    s_ref[1] = s[:, HH:]


def _tc_bnrelu(y, ssum, ssq, dq, g, be):
    return pl.pallas_call(
        _bnrelu_body,
        grid=(NG,),
        in_specs=[
            pl.BlockSpec((BN_R, H), lambda i: (i, 0)),
            pl.BlockSpec((1, H), lambda i: (0, 0)),
            pl.BlockSpec((1, H), lambda i: (0, 0)),
            pl.BlockSpec((BN_R, 1), lambda i: (i, 0)),
            pl.BlockSpec((1, H), lambda i: (0, 0)),
            pl.BlockSpec((1, H), lambda i: (0, 0)),
        ],
        out_specs=[pl.BlockSpec((NC, BN_R, HH), lambda i: (0, i, 0))],
        out_shape=[jax.ShapeDtypeStruct((NC, NP, HH), jnp.float32)],
    )(y, ssum, ssq, dq, g, be)


def _pool_body(a_ref, dq_ref, wt_ref, wb_ref, b_ref, batch_ref,
               psum_ref, pcnt_ref):
    i = pl.program_id(0)
    dq = dq_ref[...]
    y = (jnp.dot(a_ref[0] * dq, wt_ref[...], preferred_element_type=jnp.float32)
         + jnp.dot(a_ref[1] * dq, wb_ref[...], preferred_element_type=jnp.float32)
         + b_ref[...])
    rows = i * BN_R + lax.broadcasted_iota(jnp.int32, (BN_R, 1), 0)
    seg = lax.broadcasted_iota(jnp.int32, (BN_R, B), 1)
    oh = jnp.where((batch_ref[...] == seg) & (rows < N), 1.0, 0.0)

    @pl.when(i == 0)
    def _():
        psum_ref[...] = jnp.zeros_like(psum_ref)
        pcnt_ref[...] = jnp.zeros_like(pcnt_ref)

    psum_ref[...] += lax.dot_general(
        oh, y, (((0,), (0,)), ((), ())), preferred_element_type=jnp.float32)
    pcnt_ref[...] += lax.dot_general(
        oh, jnp.ones((BN_R, 1), jnp.float32), (((0,), (0,)), ((), ())),
        preferred_element_type=jnp.float32)


def _tc_pool(agg, dq, wt, wb, b, batch):
    return pl.pallas_call(
        _pool_body,
        grid=(NG,),
        in_specs=[
            pl.BlockSpec((NC, BN_R, HH), lambda i: (0, i, 0)),
            pl.BlockSpec((BN_R, 1), lambda i: (i, 0)),
            pl.BlockSpec((HH, H), lambda i: (0, 0)),
            pl.BlockSpec((HH, H), lambda i: (0, 0)),
            pl.BlockSpec((1, H), lambda i: (0, 0)),
            pl.BlockSpec((BN_R, 1), lambda i: (i, 0)),
        ],
        out_specs=[
            pl.BlockSpec((B, H), lambda i: (0, 0)),
            pl.BlockSpec((B, 1), lambda i: (0, 0)),
        ],
        out_shape=[
            jax.ShapeDtypeStruct((B, H), jnp.float32),
            jax.ShapeDtypeStruct((B, 1), jnp.float32),
        ],
        compiler_params=pltpu.CompilerParams(
            dimension_semantics=("arbitrary",)),
    )(agg, dq, wt, wb, b, batch)


def _head_body(psum_ref, pcnt_ref, gf_ref, gw_ref, gb_ref,
               w1a_ref, w1b_ref, b1_ref, w2_ref, b2_ref, out_ref):
    pooled = psum_ref[...] / jnp.maximum(pcnt_ref[...], 1.0)
    grepr = jnp.maximum(
        jnp.dot(gf_ref[...], gw_ref[...], preferred_element_type=jnp.float32)
        + gb_ref[...], 0.0)
    hid = jnp.maximum(
        jnp.dot(pooled, w1a_ref[...], preferred_element_type=jnp.float32)
        + jnp.dot(grepr, w1b_ref[...], preferred_element_type=jnp.float32)
        + b1_ref[...], 0.0)
    out_ref[...] = (
        jnp.dot(hid, w2_ref[...], preferred_element_type=jnp.float32)
        + b2_ref[...])


def _tc_head(psum, pcnt, gf, gw, gb, w1a, w1b, b1, w2p, b2p):
    return pl.pallas_call(
        _head_body,
        out_shape=jax.ShapeDtypeStruct((B, 8), jnp.float32),
    )(psum, pcnt, gf, gw, gb, w1a, w1b, b1, w2p, b2p)


def kernel(x, edge_index, batch, global_features, W1, b1, W2, b2, W3, b3,
           g1, be1, g2, be2, gW, gb, pW1, pb1, pW2, pb2):
    f32 = jnp.float32
    pad_e = EP - E
    src_p = jnp.concatenate(
        [edge_index[0], jnp.full((pad_e,), TRASH, jnp.int32)]
    ).reshape(16, NROW, CH)
    dst_flat = jnp.concatenate(
        [edge_index[1], jnp.full((pad_e,), TRASH, jnp.int32)])
    dst_p = dst_flat.reshape(16, NROW, CH)
    dst_deg = dst_flat.reshape(NC, 16, DROW, CH)

    x_p = jnp.concatenate([x, jnp.zeros((NP - N, D), f32)], axis=0)
    batch_p = jnp.concatenate(
        [batch, jnp.zeros((NP - N,), jnp.int32)]).reshape(NP, 1)
    zeros_np = jnp.zeros((NP,), f32)

    w1t, w1b_ = W1[:HH], W1[HH:]
    w2t, w2b_ = W2[:HH], W2[HH:]
    w3t, w3b_ = W3[:HH], W3[HH:]
    b1r, b2r, b3r = b1.reshape(1, H), b2.reshape(1, H), b3.reshape(1, H)
    g1r, be1r = g1.reshape(1, H), be1.reshape(1, H)
    g2r, be2r = g2.reshape(1, H), be2.reshape(1, H)
    gbr = gb.reshape(1, G)
    pW1a, pW1b = pW1[:H], pW1[H:]
    pb1r = pb1.reshape(1, H)
    pW2p = jnp.concatenate([pW2, jnp.zeros((H, 8 - T), f32)], axis=1)
    pb2p = jnp.concatenate([pb2, jnp.zeros((8 - T,), f32)]).reshape(1, 8)

    p = _sc_degree(dst_deg, zeros_np)
    dq, s = _tc_prep(p, x_p)

    agg = _sc_aggregate(s, src_p, dst_p)
    y, ssum, ssq = _tc_matstats(agg, dq, w1t, w1b_, b1r)
    s = _tc_bnrelu(y, ssum, ssq, dq, g1r, be1r)[0]

    agg = _sc_aggregate(s, src_p, dst_p)
    y, ssum, ssq = _tc_matstats(agg, dq, w2t, w2b_, b2r)
    s = _tc_bnrelu(y, ssum, ssq, dq, g2r, be2r)[0]

    agg = _sc_aggregate(s, src_p, dst_p)
    psum, pcnt = _tc_pool(agg, dq, w3t, w3b_, b3r, batch_p)

    out = _tc_head(psum, pcnt, global_features, gW, gbr,
                   pW1a, pW1b, pb1r, pW2p, pb2p)
    return out[:, :T]


# speedup vs baseline: 22.3597x; 1.1097x over previous
import functools

import jax
import jax.numpy as jnp
from jax import lax
from jax.experimental import pallas as pl
from jax.experimental.pallas import tpu as pltpu
from jax.experimental.pallas import tpu_sc as plsc

N = 100000
E = 1600000
D = 32
H = 32
G = 16
B = 128
T = 5

NC = 2
NS = 16
HH = H // 2

NP = 100352
TRASH = N
EP = 1605632
CH = 128
KJ = 4
KD = 8
NROW = EP // (16 * CH)
NMACRO = NROW // KJ
DROW = EP // (2 * 16 * CH)
DMACRO = DROW // KD
SL = NP // 16

_MESH = plsc.VectorSubcoreMesh(
    core_axis_name="c", subcore_axis_name="s", num_cores=NC, num_subcores=NS)


@functools.partial(
    pl.kernel,
    out_type=jax.ShapeDtypeStruct((NC, NP), jnp.float32),
    mesh=_MESH,
    scratch_types=[
        pltpu.VMEM_SHARED((NP,), jnp.float32),
        pltpu.VMEM((KD, CH), jnp.int32),
        pltpu.VMEM((CH,), jnp.float32),
        pltpu.SemaphoreType.DMA,
    ],
    compiler_params=pltpu.CompilerParams(use_tc_tiling_on_sc=False),
)
def _sc_degree(dst_hbm, zeros_hbm, out_hbm, acc, dbuf, onesv, ssem):
    c = lax.axis_index("c")
    t = lax.axis_index("s")
    pltpu.sync_copy(zeros_hbm.at[pl.ds(t * SL, SL)], acc.at[pl.ds(t * SL, SL)])
    for i in range(CH // 16):
        onesv[pl.ds(i * 16, 16)] = jnp.full((16,), 1.0, jnp.float32)
    plsc.subcore_barrier()

    def macro(kb, carry):
        pltpu.sync_copy(dst_hbm.at[c, t, pl.ds(kb * KD, KD)], dbuf)
        descs = [
            pltpu.async_copy(onesv, acc.at[dbuf.at[j]], ssem, add=True)
            for j in range(KD)
        ]
        for d in descs:
            d.wait()
        return carry

    lax.fori_loop(0, DMACRO, macro, 0)
    plsc.subcore_barrier()
    pltpu.sync_copy(acc.at[pl.ds(t * SL, SL)], out_hbm.at[c, pl.ds(t * SL, SL)])


@functools.partial(
    pl.kernel,
    out_type=jax.ShapeDtypeStruct((NC, NP, HH), jnp.float32),
    mesh=_MESH,
    scratch_types=[
        pltpu.VMEM_SHARED((NP, HH), jnp.float32),
        pltpu.VMEM((2, KJ, CH), jnp.int32),
        pltpu.VMEM((2, KJ, CH), jnp.int32),
        pltpu.VMEM((2, KJ, CH, HH), jnp.float32),
        pltpu.SemaphoreType.DMA,
        pltpu.SemaphoreType.DMA,
    ],
    compiler_params=pltpu.CompilerParams(use_tc_tiling_on_sc=False),
)
def _sc_aggregate(s_hbm, src_hbm, dst_hbm, out_hbm,
                  acc, sbuf, dbuf, rows, gsem, ssem):
    c = lax.axis_index("c")
    t = lax.axis_index("s")
    pltpu.sync_copy(s_hbm.at[c, pl.ds(t * SL, SL)], acc.at[pl.ds(t * SL, SL)])
    plsc.subcore_barrier()

    def load_idx(kb, p):
        pltpu.sync_copy(src_hbm.at[t, pl.ds(kb * KJ, KJ)], sbuf.at[p])
        pltpu.sync_copy(dst_hbm.at[t, pl.ds(kb * KJ, KJ)], dbuf.at[p])

    def fire_gathers(p):
        for j in range(KJ):
            pltpu.async_copy(s_hbm.at[c].at[sbuf.at[p, j]], rows.at[p, j],
                             gsem)

    def scatter_sync(p):
        sd = [
            pltpu.async_copy(rows.at[p, j], acc.at[dbuf.at[p, j]], ssem,
                             add=True)
            for j in range(KJ)
        ]
        for d in sd:
            d.wait()

    def wait_gathers(p):
        for j in range(KJ):
            pltpu.make_async_copy(s_hbm.at[c].at[sbuf.at[p, j]],
                                  rows.at[p, j], gsem).wait()

    load_idx(0, 0)
    fire_gathers(0)

    def macro2(kb2, carry):
        kb = kb2 * 2
        load_idx(kb + 1, 1)
        wait_gathers(0)
        fire_gathers(1)
        scatter_sync(0)

        @pl.when(kb2 < NMACRO // 2 - 1)
        def _():
            load_idx(kb + 2, 0)

        wait_gathers(1)

        @pl.when(kb2 < NMACRO // 2 - 1)
        def _():
            fire_gathers(0)

        scatter_sync(1)
        return carry

    lax.fori_loop(0, NMACRO // 2, macro2, 0)
    plsc.subcore_barrier()
    pltpu.sync_copy(acc.at[pl.ds(t * SL, SL)], out_hbm.at[c, pl.ds(t * SL, SL)])


BN_R = 1024
NG = NP // BN_R


def _prep_body(p_ref, x_ref, dq_ref, s_ref):
    total = p_ref[0] + p_ref[1]
    dq = lax.rsqrt(1.0 + total)
    dq_ref[...] = dq
    s = x_ref[...] * dq
    s_ref[0] = s[:, :HH]
    s_ref[1] = s[:, HH:]


def _tc_prep(p, x):
    return pl.pallas_call(
        _prep_body,
        grid=(NG,),
        in_specs=[
            pl.BlockSpec((NC, BN_R, 1), lambda i: (0, i, 0)),
            pl.BlockSpec((BN_R, D), lambda i: (i, 0)),
        ],
        out_specs=[
            pl.BlockSpec((BN_R, 1), lambda i: (i, 0)),
            pl.BlockSpec((NC, BN_R, HH), lambda i: (0, i, 0)),
        ],
        out_shape=[
            jax.ShapeDtypeStruct((NP, 1), jnp.float32),
            jax.ShapeDtypeStruct((NC, NP, HH), jnp.float32),
        ],
    )(p.reshape(NC, NP, 1), x)


def _matstats_body(a_ref, dq_ref, wt_ref, wb_ref, b_ref,
                   y_ref, ssum_ref, ssq_ref):
    i = pl.program_id(0)
    dq = dq_ref[...]
    y = (jnp.dot(a_ref[0] * dq, wt_ref[...], preferred_element_type=jnp.float32)
         + jnp.dot(a_ref[1] * dq, wb_ref[...], preferred_element_type=jnp.float32)
         + b_ref[...])
    y_ref[...] = y
    rows = i * BN_R + lax.broadcasted_iota(jnp.int32, (BN_R, 1), 0)
    ym = jnp.where(rows < N, y, 0.0)

    @pl.when(i == 0)
    def _():
        ssum_ref[...] = jnp.zeros_like(ssum_ref)
        ssq_ref[...] = jnp.zeros_like(ssq_ref)

    ssum_ref[...] += jnp.sum(ym, axis=0, keepdims=True)
    ssq_ref[...] += jnp.sum(ym * ym, axis=0, keepdims=True)


def _tc_matstats(agg, dq, wt, wb, b):
    return pl.pallas_call(
        _matstats_body,
        grid=(NG,),
        in_specs=[
            pl.BlockSpec((NC, BN_R, HH), lambda i: (0, i, 0)),
            pl.BlockSpec((BN_R, 1), lambda i: (i, 0)),
            pl.BlockSpec((HH, H), lambda i: (0, 0)),
            pl.BlockSpec((HH, H), lambda i: (0, 0)),
            pl.BlockSpec((1, H), lambda i: (0, 0)),
        ],
        out_specs=[
            pl.BlockSpec((BN_R, H), lambda i: (i, 0)),
            pl.BlockSpec((1, H), lambda i: (0, 0)),
            pl.BlockSpec((1, H), lambda i: (0, 0)),
        ],
        out_shape=[
            jax.ShapeDtypeStruct((NP, H), jnp.float32),
            jax.ShapeDtypeStruct((1, H), jnp.float32),
            jax.ShapeDtypeStruct((1, H), jnp.float32),
        ],
        compiler_params=pltpu.CompilerParams(
            dimension_semantics=("arbitrary",)),
    )(agg, dq, wt, wb, b)


def _bnrelu_body(y_ref, ssum_ref, ssq_ref, dq_ref, g_ref, be_ref, s_ref):
    mean = ssum_ref[...] / N
    var = ssq_ref[...] / N - mean * mean
    inv = lax.rsqrt(var + 1e-5)
    h = jnp.maximum((y_ref[...] - mean) * inv * g_ref[...] + be_ref[...], 0.0)
    s = h * dq_ref[...]
    s_ref[0] = s[:, :HH]
    s_ref[1] = s[:, HH:]


def _tc_bnrelu(y, ssum, ssq, dq, g, be):
    return pl.pallas_call(
        _bnrelu_body,
        grid=(NG,),
        in_specs=[
            pl.BlockSpec((BN_R, H), lambda i: (i, 0)),
            pl.BlockSpec((1, H), lambda i: (0, 0)),
            pl.BlockSpec((1, H), lambda i: (0, 0)),
            pl.BlockSpec((BN_R, 1), lambda i: (i, 0)),
            pl.BlockSpec((1, H), lambda i: (0, 0)),
            pl.BlockSpec((1, H), lambda i: (0, 0)),
        ],
        out_specs=[pl.BlockSpec((NC, BN_R, HH), lambda i: (0, i, 0))],
        out_shape=[jax.ShapeDtypeStruct((NC, NP, HH), jnp.float32)],
    )(y, ssum, ssq, dq, g, be)


def _pool_body(a_ref, dq_ref, wt_ref, wb_ref, b_ref, batch_ref,
               psum_ref, pcnt_ref):
    i = pl.program_id(0)
    dq = dq_ref[...]
    y = (jnp.dot(a_ref[0] * dq, wt_ref[...], preferred_element_type=jnp.float32)
         + jnp.dot(a_ref[1] * dq, wb_ref[...], preferred_element_type=jnp.float32)
         + b_ref[...])
    rows = i * BN_R + lax.broadcasted_iota(jnp.int32, (BN_R, 1), 0)
    seg = lax.broadcasted_iota(jnp.int32, (BN_R, B), 1)
    oh = jnp.where((batch_ref[...] == seg) & (rows < N), 1.0, 0.0)

    @pl.when(i == 0)
    def _():
        psum_ref[...] = jnp.zeros_like(psum_ref)
        pcnt_ref[...] = jnp.zeros_like(pcnt_ref)

    psum_ref[...] += lax.dot_general(
        oh, y, (((0,), (0,)), ((), ())), preferred_element_type=jnp.float32)
    pcnt_ref[...] += lax.dot_general(
        oh, jnp.ones((BN_R, 1), jnp.float32), (((0,), (0,)), ((), ())),
        preferred_element_type=jnp.float32)


def _tc_pool(agg, dq, wt, wb, b, batch):
    return pl.pallas_call(
        _pool_body,
        grid=(NG,),
        in_specs=[
            pl.BlockSpec((NC, BN_R, HH), lambda i: (0, i, 0)),
            pl.BlockSpec((BN_R, 1), lambda i: (i, 0)),
            pl.BlockSpec((HH, H), lambda i: (0, 0)),
            pl.BlockSpec((HH, H), lambda i: (0, 0)),
            pl.BlockSpec((1, H), lambda i: (0, 0)),
            pl.BlockSpec((BN_R, 1), lambda i: (i, 0)),
        ],
        out_specs=[
            pl.BlockSpec((B, H), lambda i: (0, 0)),
            pl.BlockSpec((B, 1), lambda i: (0, 0)),
        ],
        out_shape=[
            jax.ShapeDtypeStruct((B, H), jnp.float32),
            jax.ShapeDtypeStruct((B, 1), jnp.float32),
        ],
        compiler_params=pltpu.CompilerParams(
            dimension_semantics=("arbitrary",)),
    )(agg, dq, wt, wb, b, batch)


def _head_body(psum_ref, pcnt_ref, gf_ref, gw_ref, gb_ref,
               w1a_ref, w1b_ref, b1_ref, w2_ref, b2_ref, out_ref):
    pooled = psum_ref[...] / jnp.maximum(pcnt_ref[...], 1.0)
    grepr = jnp.maximum(
        jnp.dot(gf_ref[...], gw_ref[...], preferred_element_type=jnp.float32)
        + gb_ref[...], 0.0)
    hid = jnp.maximum(
        jnp.dot(pooled, w1a_ref[...], preferred_element_type=jnp.float32)
        + jnp.dot(grepr, w1b_ref[...], preferred_element_type=jnp.float32)
        + b1_ref[...], 0.0)
    out_ref[...] = (
        jnp.dot(hid, w2_ref[...], preferred_element_type=jnp.float32)
        + b2_ref[...])


def _tc_head(psum, pcnt, gf, gw, gb, w1a, w1b, b1, w2p, b2p):
    return pl.pallas_call(
        _head_body,
        out_shape=jax.ShapeDtypeStruct((B, 8), jnp.float32),
    )(psum, pcnt, gf, gw, gb, w1a, w1b, b1, w2p, b2p)


def kernel(x, edge_index, batch, global_features, W1, b1, W2, b2, W3, b3,
           g1, be1, g2, be2, gW, gb, pW1, pb1, pW2, pb2):
    f32 = jnp.float32
    pad_e = EP - E
    pad_idx = TRASH + jnp.arange(pad_e, dtype=jnp.int32) % (NP - N)
    src_p = jnp.concatenate([edge_index[0], pad_idx]).reshape(16, NROW, CH)
    dst_flat = jnp.concatenate([edge_index[1], pad_idx])
    dst_p = dst_flat.reshape(16, NROW, CH)
    dst_deg = dst_flat.reshape(NC, 16, DROW, CH)

    x_p = jnp.concatenate([x, jnp.zeros((NP - N, D), f32)], axis=0)
    batch_p = jnp.concatenate(
        [batch, jnp.zeros((NP - N,), jnp.int32)]).reshape(NP, 1)
    zeros_np = jnp.zeros((NP,), f32)

    w1t, w1b_ = W1[:HH], W1[HH:]
    w2t, w2b_ = W2[:HH], W2[HH:]
    w3t, w3b_ = W3[:HH], W3[HH:]
    b1r, b2r, b3r = b1.reshape(1, H), b2.reshape(1, H), b3.reshape(1, H)
    g1r, be1r = g1.reshape(1, H), be1.reshape(1, H)
    g2r, be2r = g2.reshape(1, H), be2.reshape(1, H)
    gbr = gb.reshape(1, G)
    pW1a, pW1b = pW1[:H], pW1[H:]
    pb1r = pb1.reshape(1, H)
    pW2p = jnp.concatenate([pW2, jnp.zeros((H, 8 - T), f32)], axis=1)
    pb2p = jnp.concatenate([pb2, jnp.zeros((8 - T,), f32)]).reshape(1, 8)

    p = _sc_degree(dst_deg, zeros_np)
    dq, s = _tc_prep(p, x_p)

    agg = _sc_aggregate(s, src_p, dst_p)
    y, ssum, ssq = _tc_matstats(agg, dq, w1t, w1b_, b1r)
    s = _tc_bnrelu(y, ssum, ssq, dq, g1r, be1r)[0]

    agg = _sc_aggregate(s, src_p, dst_p)
    y, ssum, ssq = _tc_matstats(agg, dq, w2t, w2b_, b2r)
    s = _tc_bnrelu(y, ssum, ssq, dq, g2r, be2r)[0]

    agg = _sc_aggregate(s, src_p, dst_p)
    psum, pcnt = _tc_pool(agg, dq, w3t, w3b_, b3r, batch_p)

    out = _tc_head(psum, pcnt, global_features, gW, gbr,
                   pW1a, pW1b, pb1r, pW2p, pb2p)
    return out[:, :T]


# speedup vs baseline: 22.4320x; 1.0032x over previous
import functools

import jax
import jax.numpy as jnp
from jax import lax
from jax.experimental import pallas as pl
from jax.experimental.pallas import tpu as pltpu
from jax.experimental.pallas import tpu_sc as plsc

N = 100000
E = 1600000
D = 32
H = 32
G = 16
B = 128
T = 5

NC = 2
NS = 16
HH = H // 2

NP = 100352
TRASH = N
EP = 1605632
CH = 512
KJ = 1
KD = 8
NROW = EP // (16 * CH)
NMACRO = NROW // KJ
DCH = 128
DROW = EP // (2 * 16 * DCH)
DMACRO = DROW // KD
SL = NP // 16

_MESH = plsc.VectorSubcoreMesh(
    core_axis_name="c", subcore_axis_name="s", num_cores=NC, num_subcores=NS)


@functools.partial(
    pl.kernel,
    out_type=jax.ShapeDtypeStruct((NC, NP), jnp.float32),
    mesh=_MESH,
    scratch_types=[
        pltpu.VMEM_SHARED((NP,), jnp.float32),
        pltpu.VMEM((KD, DCH), jnp.int32),
        pltpu.VMEM((DCH,), jnp.float32),
        pltpu.SemaphoreType.DMA,
    ],
    compiler_params=pltpu.CompilerParams(use_tc_tiling_on_sc=False),
)
def _sc_degree(dst_hbm, zeros_hbm, out_hbm, acc, dbuf, onesv, ssem):
    c = lax.axis_index("c")
    t = lax.axis_index("s")
    pltpu.sync_copy(zeros_hbm.at[pl.ds(t * SL, SL)], acc.at[pl.ds(t * SL, SL)])
    for i in range(DCH // 16):
        onesv[pl.ds(i * 16, 16)] = jnp.full((16,), 1.0, jnp.float32)
    plsc.subcore_barrier()

    def macro(kb, carry):
        pltpu.sync_copy(dst_hbm.at[c, t, pl.ds(kb * KD, KD)], dbuf)
        descs = [
            pltpu.async_copy(onesv, acc.at[dbuf.at[j]], ssem, add=True)
            for j in range(KD)
        ]
        for d in descs:
            d.wait()
        return carry

    lax.fori_loop(0, DMACRO, macro, 0)
    plsc.subcore_barrier()
    pltpu.sync_copy(acc.at[pl.ds(t * SL, SL)], out_hbm.at[c, pl.ds(t * SL, SL)])


@functools.partial(
    pl.kernel,
    out_type=jax.ShapeDtypeStruct((NC, NP, HH), jnp.float32),
    mesh=_MESH,
    scratch_types=[
        pltpu.VMEM_SHARED((NP, HH), jnp.float32),
        pltpu.VMEM((2, KJ, CH), jnp.int32),
        pltpu.VMEM((2, KJ, CH), jnp.int32),
        pltpu.VMEM((2, KJ, CH, HH), jnp.float32),
        pltpu.SemaphoreType.DMA,
        pltpu.SemaphoreType.DMA,
    ],
    compiler_params=pltpu.CompilerParams(use_tc_tiling_on_sc=False),
)
def _sc_aggregate(s_hbm, src_hbm, dst_hbm, out_hbm,
                  acc, sbuf, dbuf, rows, gsem, ssem):
    c = lax.axis_index("c")
    t = lax.axis_index("s")
    pltpu.sync_copy(s_hbm.at[c, pl.ds(t * SL, SL)], acc.at[pl.ds(t * SL, SL)])
    plsc.subcore_barrier()

    def load_idx(kb, p):
        pltpu.sync_copy(src_hbm.at[t, pl.ds(kb * KJ, KJ)], sbuf.at[p])
        pltpu.sync_copy(dst_hbm.at[t, pl.ds(kb * KJ, KJ)], dbuf.at[p])

    def fire_gathers(p):
        for j in range(KJ):
            pltpu.async_copy(s_hbm.at[c].at[sbuf.at[p, j]], rows.at[p, j],
                             gsem)

    def scatter_sync(p):
        sd = [
            pltpu.async_copy(rows.at[p, j], acc.at[dbuf.at[p, j]], ssem,
                             add=True)
            for j in range(KJ)
        ]
        for d in sd:
            d.wait()

    def wait_gathers(p):
        for j in range(KJ):
            pltpu.make_async_copy(s_hbm.at[c].at[sbuf.at[p, j]],
                                  rows.at[p, j], gsem).wait()

    load_idx(0, 0)
    fire_gathers(0)

    def macro2(kb2, carry):
        kb = kb2 * 2
        load_idx(kb + 1, 1)
        wait_gathers(0)
        fire_gathers(1)
        scatter_sync(0)

        @pl.when(kb2 < NMACRO // 2 - 1)
        def _():
            load_idx(kb + 2, 0)

        wait_gathers(1)

        @pl.when(kb2 < NMACRO // 2 - 1)
        def _():
            fire_gathers(0)

        scatter_sync(1)
        return carry

    lax.fori_loop(0, NMACRO // 2, macro2, 0)
    plsc.subcore_barrier()
    pltpu.sync_copy(acc.at[pl.ds(t * SL, SL)], out_hbm.at[c, pl.ds(t * SL, SL)])


BN_R = 1024
NG = NP // BN_R


def _prep_body(p_ref, x_ref, dq_ref, s_ref):
    total = p_ref[0] + p_ref[1]
    dq = lax.rsqrt(1.0 + total)
    dq_ref[...] = dq
    s = x_ref[...] * dq
    s_ref[0] = s[:, :HH]
    s_ref[1] = s[:, HH:]


def _tc_prep(p, x):
    return pl.pallas_call(
        _prep_body,
        grid=(NG,),
        in_specs=[
            pl.BlockSpec((NC, BN_R, 1), lambda i: (0, i, 0)),
            pl.BlockSpec((BN_R, D), lambda i: (i, 0)),
        ],
        out_specs=[
            pl.BlockSpec((BN_R, 1), lambda i: (i, 0)),
            pl.BlockSpec((NC, BN_R, HH), lambda i: (0, i, 0)),
        ],
        out_shape=[
            jax.ShapeDtypeStruct((NP, 1), jnp.float32),
            jax.ShapeDtypeStruct((NC, NP, HH), jnp.float32),
        ],
    )(p.reshape(NC, NP, 1), x)


def _matstats_body(a_ref, dq_ref, wt_ref, wb_ref, b_ref,
                   y_ref, ssum_ref, ssq_ref):
    i = pl.program_id(0)
    dq = dq_ref[...]
    y = (jnp.dot(a_ref[0] * dq, wt_ref[...], preferred_element_type=jnp.float32)
         + jnp.dot(a_ref[1] * dq, wb_ref[...], preferred_element_type=jnp.float32)
         + b_ref[...])
    y_ref[...] = y
    rows = i * BN_R + lax.broadcasted_iota(jnp.int32, (BN_R, 1), 0)
    ym = jnp.where(rows < N, y, 0.0)

    @pl.when(i == 0)
    def _():
        ssum_ref[...] = jnp.zeros_like(ssum_ref)
        ssq_ref[...] = jnp.zeros_like(ssq_ref)

    ssum_ref[...] += jnp.sum(ym, axis=0, keepdims=True)
    ssq_ref[...] += jnp.sum(ym * ym, axis=0, keepdims=True)


def _tc_matstats(agg, dq, wt, wb, b):
    return pl.pallas_call(
        _matstats_body,
        grid=(NG,),
        in_specs=[
            pl.BlockSpec((NC, BN_R, HH), lambda i: (0, i, 0)),
            pl.BlockSpec((BN_R, 1), lambda i: (i, 0)),
            pl.BlockSpec((HH, H), lambda i: (0, 0)),
            pl.BlockSpec((HH, H), lambda i: (0, 0)),
            pl.BlockSpec((1, H), lambda i: (0, 0)),
        ],
        out_specs=[
            pl.BlockSpec((BN_R, H), lambda i: (i, 0)),
            pl.BlockSpec((1, H), lambda i: (0, 0)),
            pl.BlockSpec((1, H), lambda i: (0, 0)),
        ],
        out_shape=[
            jax.ShapeDtypeStruct((NP, H), jnp.float32),
            jax.ShapeDtypeStruct((1, H), jnp.float32),
            jax.ShapeDtypeStruct((1, H), jnp.float32),
        ],
        compiler_params=pltpu.CompilerParams(
            dimension_semantics=("arbitrary",)),
    )(agg, dq, wt, wb, b)


def _bnrelu_body(y_ref, ssum_ref, ssq_ref, dq_ref, g_ref, be_ref, s_ref):
    mean = ssum_ref[...] / N
    var = ssq_ref[...] / N - mean * mean
    inv = lax.rsqrt(var + 1e-5)
    h = jnp.maximum((y_ref[...] - mean) * inv * g_ref[...] + be_ref[...], 0.0)
    s = h * dq_ref[...]
    s_ref[0] = s[:, :HH]
    s_ref[1] = s[:, HH:]


def _tc_bnrelu(y, ssum, ssq, dq, g, be):
    return pl.pallas_call(
        _bnrelu_body,
        grid=(NG,),
        in_specs=[
            pl.BlockSpec((BN_R, H), lambda i: (i, 0)),
            pl.BlockSpec((1, H), lambda i: (0, 0)),
            pl.BlockSpec((1, H), lambda i: (0, 0)),
            pl.BlockSpec((BN_R, 1), lambda i: (i, 0)),
            pl.BlockSpec((1, H), lambda i: (0, 0)),
            pl.BlockSpec((1, H), lambda i: (0, 0)),
        ],
        out_specs=[pl.BlockSpec((NC, BN_R, HH), lambda i: (0, i, 0))],
        out_shape=[jax.ShapeDtypeStruct((NC, NP, HH), jnp.float32)],
    )(y, ssum, ssq, dq, g, be)


def _pool_body(a_ref, dq_ref, wt_ref, wb_ref, b_ref, batch_ref,
               psum_ref, pcnt_ref):
    i = pl.program_id(0)
    dq = dq_ref[...]
    y = (jnp.dot(a_ref[0] * dq, wt_ref[...], preferred_element_type=jnp.float32)
         + jnp.dot(a_ref[1] * dq, wb_ref[...], preferred_element_type=jnp.float32)
         + b_ref[...])
    rows = i * BN_R + lax.broadcasted_iota(jnp.int32, (BN_R, 1), 0)
    seg = lax.broadcasted_iota(jnp.int32, (BN_R, B), 1)
    oh = jnp.where((batch_ref[...] == seg) & (rows < N), 1.0, 0.0)

    @pl.when(i == 0)
    def _():
        psum_ref[...] = jnp.zeros_like(psum_ref)
        pcnt_ref[...] = jnp.zeros_like(pcnt_ref)

    psum_ref[...] += lax.dot_general(
        oh, y, (((0,), (0,)), ((), ())), preferred_element_type=jnp.float32)
    pcnt_ref[...] += lax.dot_general(
        oh, jnp.ones((BN_R, 1), jnp.float32), (((0,), (0,)), ((), ())),
        preferred_element_type=jnp.float32)


def _tc_pool(agg, dq, wt, wb, b, batch):
    return pl.pallas_call(
        _pool_body,
        grid=(NG,),
        in_specs=[
            pl.BlockSpec((NC, BN_R, HH), lambda i: (0, i, 0)),
            pl.BlockSpec((BN_R, 1), lambda i: (i, 0)),
            pl.BlockSpec((HH, H), lambda i: (0, 0)),
            pl.BlockSpec((HH, H), lambda i: (0, 0)),
            pl.BlockSpec((1, H), lambda i: (0, 0)),
            pl.BlockSpec((BN_R, 1), lambda i: (i, 0)),
        ],
        out_specs=[
            pl.BlockSpec((B, H), lambda i: (0, 0)),
            pl.BlockSpec((B, 1), lambda i: (0, 0)),
        ],
        out_shape=[
            jax.ShapeDtypeStruct((B, H), jnp.float32),
            jax.ShapeDtypeStruct((B, 1), jnp.float32),
        ],
        compiler_params=pltpu.CompilerParams(
            dimension_semantics=("arbitrary",)),
    )(agg, dq, wt, wb, b, batch)


def _head_body(psum_ref, pcnt_ref, gf_ref, gw_ref, gb_ref,
               w1a_ref, w1b_ref, b1_ref, w2_ref, b2_ref, out_ref):
    pooled = psum_ref[...] / jnp.maximum(pcnt_ref[...], 1.0)
    grepr = jnp.maximum(
        jnp.dot(gf_ref[...], gw_ref[...], preferred_element_type=jnp.float32)
        + gb_ref[...], 0.0)
    hid = jnp.maximum(
        jnp.dot(pooled, w1a_ref[...], preferred_element_type=jnp.float32)
        + jnp.dot(grepr, w1b_ref[...], preferred_element_type=jnp.float32)
        + b1_ref[...], 0.0)
    out_ref[...] = (
        jnp.dot(hid, w2_ref[...], preferred_element_type=jnp.float32)
        + b2_ref[...])


def _tc_head(psum, pcnt, gf, gw, gb, w1a, w1b, b1, w2p, b2p):
    return pl.pallas_call(
        _head_body,
        out_shape=jax.ShapeDtypeStruct((B, 8), jnp.float32),
    )(psum, pcnt, gf, gw, gb, w1a, w1b, b1, w2p, b2p)


def kernel(x, edge_index, batch, global_features, W1, b1, W2, b2, W3, b3,
           g1, be1, g2, be2, gW, gb, pW1, pb1, pW2, pb2):
    f32 = jnp.float32
    pad_e = EP - E
    pad_idx = TRASH + jnp.arange(pad_e, dtype=jnp.int32) % (NP - N)
    src_p = jnp.concatenate([edge_index[0], pad_idx]).reshape(16, NROW, CH)
    dst_flat = jnp.concatenate([edge_index[1], pad_idx])
    dst_p = dst_flat.reshape(16, NROW, CH)
    dst_deg = dst_flat.reshape(NC, 16, DROW, DCH)

    x_p = jnp.concatenate([x, jnp.zeros((NP - N, D), f32)], axis=0)
    batch_p = jnp.concatenate(
        [batch, jnp.zeros((NP - N,), jnp.int32)]).reshape(NP, 1)
    zeros_np = jnp.zeros((NP,), f32)

    w1t, w1b_ = W1[:HH], W1[HH:]
    w2t, w2b_ = W2[:HH], W2[HH:]
    w3t, w3b_ = W3[:HH], W3[HH:]
    b1r, b2r, b3r = b1.reshape(1, H), b2.reshape(1, H), b3.reshape(1, H)
    g1r, be1r = g1.reshape(1, H), be1.reshape(1, H)
    g2r, be2r = g2.reshape(1, H), be2.reshape(1, H)
    gbr = gb.reshape(1, G)
    pW1a, pW1b = pW1[:H], pW1[H:]
    pb1r = pb1.reshape(1, H)
    pW2p = jnp.concatenate([pW2, jnp.zeros((H, 8 - T), f32)], axis=1)
    pb2p = jnp.concatenate([pb2, jnp.zeros((8 - T,), f32)]).reshape(1, 8)

    p = _sc_degree(dst_deg, zeros_np)
    dq, s = _tc_prep(p, x_p)

    agg = _sc_aggregate(s, src_p, dst_p)
    y, ssum, ssq = _tc_matstats(agg, dq, w1t, w1b_, b1r)
    s = _tc_bnrelu(y, ssum, ssq, dq, g1r, be1r)[0]

    agg = _sc_aggregate(s, src_p, dst_p)
    y, ssum, ssq = _tc_matstats(agg, dq, w2t, w2b_, b2r)
    s = _tc_bnrelu(y, ssum, ssq, dq, g2r, be2r)[0]

    agg = _sc_aggregate(s, src_p, dst_p)
    psum, pcnt = _tc_pool(agg, dq, w3t, w3b_, b3r, batch_p)

    out = _tc_head(psum, pcnt, global_features, gW, gbr,
                   pW1a, pW1b, pb1r, pW2p, pb2p)
    return out[:, :T]


# speedup vs baseline: 37.4889x; 1.6712x over previous
import functools

import jax
import jax.numpy as jnp
from jax import lax
from jax.experimental import pallas as pl
from jax.experimental.pallas import tpu as pltpu
from jax.experimental.pallas import tpu_sc as plsc

N = 100000
E = 1600000
D = 32
H = 32
G = 16
B = 128
T = 5

NC = 2
NS = 16
HH = H // 2

NP = 100352
NP8 = NP // 8
TRASH = N
EP = 1605632
CH = 512
KJ = 1
KD = 8
DCH = 128
NROW = EP // (16 * CH)
NMACRO = NROW // KJ
DROW = EP // (2 * 16 * DCH)
DMACRO = DROW // KD
SL = NP // 16

_MESH = plsc.VectorSubcoreMesh(
    core_axis_name="c", subcore_axis_name="s", num_cores=NC, num_subcores=NS)


@functools.partial(
    pl.kernel,
    out_type=jax.ShapeDtypeStruct((NC, NP, HH), jnp.float32),
    mesh=_MESH,
    scratch_types=[
        pltpu.VMEM_SHARED((NP, HH), jnp.float32),
        pltpu.VMEM((KD, DCH), jnp.int32),
        pltpu.VMEM((DCH, HH), jnp.float32),
        pltpu.SemaphoreType.DMA,
    ],
    compiler_params=pltpu.CompilerParams(use_tc_tiling_on_sc=False),
)
def _sc_degree(dst_hbm, zeros_hbm, out_hbm, acc, dbuf, onesv, ssem):
    c = lax.axis_index("c")
    t = lax.axis_index("s")
    pltpu.sync_copy(zeros_hbm.at[pl.ds(t * SL, SL)], acc.at[pl.ds(t * SL, SL)])
    for i in range(DCH):
        onesv[i] = jnp.full((HH,), 1.0, jnp.float32)
    plsc.subcore_barrier()

    def macro(kb, carry):
        pltpu.sync_copy(dst_hbm.at[c, t, pl.ds(kb * KD, KD)], dbuf)
        descs = [
            pltpu.async_copy(onesv, acc.at[dbuf.at[j]], ssem, add=True)
            for j in range(KD)
        ]
        for d in descs:
            d.wait()
        return carry

    lax.fori_loop(0, DMACRO, macro, 0)
    plsc.subcore_barrier()
    pltpu.sync_copy(acc.at[pl.ds(t * SL, SL)], out_hbm.at[c, pl.ds(t * SL, SL)])


@functools.partial(
    pl.kernel,
    out_type=jax.ShapeDtypeStruct((NC, NP, HH), jnp.float32),
    mesh=_MESH,
    scratch_types=[
        pltpu.VMEM_SHARED((NP, HH), jnp.float32),
        pltpu.VMEM((2, KJ, CH), jnp.int32),
        pltpu.VMEM((2, KJ, CH), jnp.int32),
        pltpu.VMEM((2, KJ, CH, HH), jnp.float32),
        pltpu.SemaphoreType.DMA,
        pltpu.SemaphoreType.DMA,
    ],
    compiler_params=pltpu.CompilerParams(use_tc_tiling_on_sc=False),
)
def _sc_aggregate(s_hbm, src_hbm, dst_hbm, out_hbm,
                  acc, sbuf, dbuf, rows, gsem, ssem):
    c = lax.axis_index("c")
    t = lax.axis_index("s")
    pltpu.sync_copy(s_hbm.at[c, pl.ds(t * SL, SL)], acc.at[pl.ds(t * SL, SL)])
    plsc.subcore_barrier()

    def load_idx(kb, p):
        pltpu.sync_copy(src_hbm.at[t, pl.ds(kb * KJ, KJ)], sbuf.at[p])
        pltpu.sync_copy(dst_hbm.at[t, pl.ds(kb * KJ, KJ)], dbuf.at[p])

    def fire_gathers(p):
        for j in range(KJ):
            pltpu.async_copy(s_hbm.at[c].at[sbuf.at[p, j]], rows.at[p, j],
                             gsem)

    def wait_gathers(p):
        for j in range(KJ):
            pltpu.make_async_copy(s_hbm.at[c].at[sbuf.at[p, j]],
                                  rows.at[p, j], gsem).wait()

    def scatter_sync(p):
        sd = [
            pltpu.async_copy(rows.at[p, j], acc.at[dbuf.at[p, j]], ssem,
                             add=True)
            for j in range(KJ)
        ]
        for d in sd:
            d.wait()

    load_idx(0, 0)
    fire_gathers(0)

    def macro2(kb2, carry):
        kb = kb2 * 2
        load_idx(kb + 1, 1)
        wait_gathers(0)
        fire_gathers(1)
        scatter_sync(0)

        @pl.when(kb2 < NMACRO // 2 - 1)
        def _():
            load_idx(kb + 2, 0)

        wait_gathers(1)

        @pl.when(kb2 < NMACRO // 2 - 1)
        def _():
            fire_gathers(0)

        scatter_sync(1)
        return carry

    lax.fori_loop(0, NMACRO // 2, macro2, 0)
    plsc.subcore_barrier()
    pltpu.sync_copy(acc.at[pl.ds(t * SL, SL)], out_hbm.at[c, pl.ds(t * SL, SL)])


BN_R = 14336
BN8 = BN_R // 8
NG = NP // BN_R


def _prep_body(p_ref, x_ref, xexp_ref, s0m_ref, s1m_ref,
               dqh_ref, dq2_ref, s_ref):
    cnt = p_ref[0] + p_ref[1]
    dqh = lax.rsqrt(1.0 + cnt)
    dqh_ref[...] = dqh
    dq2 = jnp.dot(dqh, xexp_ref[...], preferred_element_type=jnp.float32, precision=lax.Precision.HIGHEST)
    dq2_ref[...] = dq2
    s = x_ref[...] * dq2
    s_ref[0] = jnp.dot(s, s0m_ref[...], preferred_element_type=jnp.float32, precision=lax.Precision.HIGHEST)
    s_ref[1] = jnp.dot(s, s1m_ref[...], preferred_element_type=jnp.float32, precision=lax.Precision.HIGHEST)


def _tc_prep(p, x, xexp, s0m, s1m):
    return pl.pallas_call(
        _prep_body,
        grid=(NG,),
        in_specs=[
            pl.BlockSpec((NC, BN8, 128), lambda i: (0, i, 0)),
            pl.BlockSpec((BN8, 256), lambda i: (i, 0)),
            pl.BlockSpec((128, 256), lambda i: (0, 0)),
            pl.BlockSpec((256, 128), lambda i: (0, 0)),
            pl.BlockSpec((256, 128), lambda i: (0, 0)),
        ],
        out_specs=[
            pl.BlockSpec((BN8, 128), lambda i: (i, 0)),
            pl.BlockSpec((BN8, 256), lambda i: (i, 0)),
            pl.BlockSpec((NC, BN8, 128), lambda i: (0, i, 0)),
        ],
        out_shape=[
            jax.ShapeDtypeStruct((NP8, 128), jnp.float32),
            jax.ShapeDtypeStruct((NP8, 256), jnp.float32),
            jax.ShapeDtypeStruct((NC, NP8, 128), jnp.float32),
        ],
    )(p, x, xexp, s0m, s1m)


def _node_mask(i, width):
    r = lax.broadcasted_iota(jnp.int32, (BN8, 256), 0)
    lane = lax.broadcasted_iota(jnp.int32, (BN8, 256), 1)
    return (i * BN_R + r * 8 + lane // width) < N


def _matstats_body(a_ref, dqh_ref, w0_ref, w1_ref, b_ref,
                   y_ref, ssum_ref, ssq_ref):
    i = pl.program_id(0)
    dqh = dqh_ref[...]
    y = (jnp.dot(a_ref[0] * dqh, w0_ref[...],
                 preferred_element_type=jnp.float32)
         + jnp.dot(a_ref[1] * dqh, w1_ref[...],
                   preferred_element_type=jnp.float32)
         + b_ref[...])
    y_ref[...] = y
    ym = jnp.where(_node_mask(i, 32), y, 0.0)

    @pl.when(i == 0)
    def _():
        ssum_ref[...] = jnp.zeros_like(ssum_ref)
        ssq_ref[...] = jnp.zeros_like(ssq_ref)

    ssum_ref[...] += jnp.sum(ym, axis=0, keepdims=True)
    ssq_ref[...] += jnp.sum(ym * ym, axis=0, keepdims=True)


def _tc_matstats(agg, dqh, w0, w1, b):
    return pl.pallas_call(
        _matstats_body,
        grid=(NG,),
        in_specs=[
            pl.BlockSpec((NC, BN8, 128), lambda i: (0, i, 0)),
            pl.BlockSpec((BN8, 128), lambda i: (i, 0)),
            pl.BlockSpec((128, 256), lambda i: (0, 0)),
            pl.BlockSpec((128, 256), lambda i: (0, 0)),
            pl.BlockSpec((1, 256), lambda i: (0, 0)),
        ],
        out_specs=[
            pl.BlockSpec((BN8, 256), lambda i: (i, 0)),
            pl.BlockSpec((1, 256), lambda i: (0, 0)),
            pl.BlockSpec((1, 256), lambda i: (0, 0)),
        ],
        out_shape=[
            jax.ShapeDtypeStruct((NP8, 256), jnp.float32),
            jax.ShapeDtypeStruct((1, 256), jnp.float32),
            jax.ShapeDtypeStruct((1, 256), jnp.float32),
        ],
        compiler_params=pltpu.CompilerParams(
            dimension_semantics=("arbitrary",)),
    )(agg, dqh, w0, w1, b)


def _bnrelu_body(y_ref, ssum_ref, ssq_ref, dq2_ref, g_ref, be_ref,
                 fold_ref, exp_ref, s0m_ref, s1m_ref, s_ref):
    fold = fold_ref[...]
    mean32 = jnp.dot(ssum_ref[...], fold,
                     preferred_element_type=jnp.float32, precision=lax.Precision.HIGHEST) / N
    var32 = jnp.dot(ssq_ref[...], fold,
                    preferred_element_type=jnp.float32, precision=lax.Precision.HIGHEST) / N - mean32 * mean32
    inv32 = lax.rsqrt(var32 + 1e-5)
    mean = jnp.dot(mean32, exp_ref[...], preferred_element_type=jnp.float32, precision=lax.Precision.HIGHEST)
    inv = jnp.dot(inv32, exp_ref[...], preferred_element_type=jnp.float32, precision=lax.Precision.HIGHEST)
    h = jnp.maximum((y_ref[...] - mean) * inv * g_ref[...] + be_ref[...], 0.0)
    s = h * dq2_ref[...]
    s_ref[0] = jnp.dot(s, s0m_ref[...], preferred_element_type=jnp.float32, precision=lax.Precision.HIGHEST)
    s_ref[1] = jnp.dot(s, s1m_ref[...], preferred_element_type=jnp.float32, precision=lax.Precision.HIGHEST)


def _tc_bnrelu(y, ssum, ssq, dq2, g, be, fold, expm, s0m, s1m):
    return pl.pallas_call(
        _bnrelu_body,
        grid=(NG,),
        in_specs=[
            pl.BlockSpec((BN8, 256), lambda i: (i, 0)),
            pl.BlockSpec((1, 256), lambda i: (0, 0)),
            pl.BlockSpec((1, 256), lambda i: (0, 0)),
            pl.BlockSpec((BN8, 256), lambda i: (i, 0)),
            pl.BlockSpec((1, 256), lambda i: (0, 0)),
            pl.BlockSpec((1, 256), lambda i: (0, 0)),
            pl.BlockSpec((256, 32), lambda i: (0, 0)),
            pl.BlockSpec((32, 256), lambda i: (0, 0)),
            pl.BlockSpec((256, 128), lambda i: (0, 0)),
            pl.BlockSpec((256, 128), lambda i: (0, 0)),
        ],
        out_specs=[pl.BlockSpec((NC, BN8, 128), lambda i: (0, i, 0))],
        out_shape=[jax.ShapeDtypeStruct((NC, NP8, 128), jnp.float32)],
    )(y, ssum, ssq, dq2, g, be, fold, expm, s0m, s1m)


def _pool_body(a_ref, dqh_ref, w0_ref, w1_ref, b_ref, bp_ref,
               psum_ref, pcnt_ref):
    dqh = dqh_ref[...]
    y = (jnp.dot(a_ref[0] * dqh, w0_ref[...],
                 preferred_element_type=jnp.float32)
         + jnp.dot(a_ref[1] * dqh, w1_ref[...],
                   preferred_element_type=jnp.float32)
         + b_ref[...])

    @pl.when(pl.program_id(0) == 0)
    def _():
        psum_ref[...] = jnp.zeros_like(psum_ref)
        pcnt_ref[...] = jnp.zeros_like(pcnt_ref)

    seg = lax.broadcasted_iota(jnp.int32, (BN8, B), 1)
    ones = jnp.ones((BN8, 1), jnp.float32)
    for k in range(8):
        ohk = jnp.where(bp_ref[:, k:k + 1] == seg, 1.0, 0.0)
        psum_ref[...] += lax.dot_general(
            ohk, y[:, 32 * k:32 * k + 32], (((0,), (0,)), ((), ())),
            preferred_element_type=jnp.float32, precision=lax.Precision.HIGHEST)
        pcnt_ref[...] += lax.dot_general(
            ohk, ones, (((0,), (0,)), ((), ())),
            preferred_element_type=jnp.float32, precision=lax.Precision.HIGHEST)


def _tc_pool(agg, dqh, w0, w1, b, bp):
    return pl.pallas_call(
        _pool_body,
        grid=(NG,),
        in_specs=[
            pl.BlockSpec((NC, BN8, 128), lambda i: (0, i, 0)),
            pl.BlockSpec((BN8, 128), lambda i: (i, 0)),
            pl.BlockSpec((128, 256), lambda i: (0, 0)),
            pl.BlockSpec((128, 256), lambda i: (0, 0)),
            pl.BlockSpec((1, 256), lambda i: (0, 0)),
            pl.BlockSpec((BN8, 8), lambda i: (i, 0)),
        ],
        out_specs=[
            pl.BlockSpec((B, H), lambda i: (0, 0)),
            pl.BlockSpec((B, 1), lambda i: (0, 0)),
        ],
        out_shape=[
            jax.ShapeDtypeStruct((B, H), jnp.float32),
            jax.ShapeDtypeStruct((B, 1), jnp.float32),
        ],
        compiler_params=pltpu.CompilerParams(
            dimension_semantics=("arbitrary",)),
    )(agg, dqh, w0, w1, b, bp)


def _head_body(psum_ref, pcnt_ref, gf_ref, gw_ref, gb_ref,
               w1a_ref, w1b_ref, b1_ref, w2_ref, b2_ref, out_ref):
    pooled = psum_ref[...] / jnp.maximum(pcnt_ref[...], 1.0)
    grepr = jnp.maximum(
        jnp.dot(gf_ref[...], gw_ref[...], preferred_element_type=jnp.float32)
        + gb_ref[...], 0.0)
    hid = jnp.maximum(
        jnp.dot(pooled, w1a_ref[...], preferred_element_type=jnp.float32)
        + jnp.dot(grepr, w1b_ref[...], preferred_element_type=jnp.float32)
        + b1_ref[...], 0.0)
    out_ref[...] = (
        jnp.dot(hid, w2_ref[...], preferred_element_type=jnp.float32)
        + b2_ref[...])


def _tc_head(psum, pcnt, gf, gw, gb, w1a, w1b, b1, w2p, b2p):
    return pl.pallas_call(
        _head_body,
        out_shape=jax.ShapeDtypeStruct((B, 8), jnp.float32),
    )(psum, pcnt, gf, gw, gb, w1a, w1b, b1, w2p, b2p)


def kernel(x, edge_index, batch, global_features, W1, b1, W2, b2, W3, b3,
           g1, be1, g2, be2, gW, gb, pW1, pb1, pW2, pb2):
    f32 = jnp.float32
    pad_e = EP - E
    pad_idx = TRASH + jnp.arange(pad_e, dtype=jnp.int32) % (NP - N)
    src_p = jnp.concatenate([edge_index[0], pad_idx]).reshape(16, NROW, CH)
    dst_flat = jnp.concatenate([edge_index[1], pad_idx])
    dst_p = dst_flat.reshape(16, NROW, CH)
    dst_deg = dst_flat.reshape(NC, 16, DROW, DCH)

    x_pk = jnp.concatenate([x, jnp.zeros((NP - N, D), f32)],
                           axis=0).reshape(NP8, 256)
    bp = jnp.concatenate(
        [batch, jnp.full((NP - N,), B, jnp.int32)]).reshape(NP8, 8)
    zeros_nph = jnp.zeros((NP, HH), f32)

    i8 = jnp.eye(8, dtype=f32)
    e16 = jnp.eye(16, dtype=f32)
    e32 = jnp.eye(32, dtype=f32)
    s0m = jnp.kron(i8, jnp.concatenate([e16, jnp.zeros((16, 16), f32)], 0))
    s1m = jnp.kron(i8, jnp.concatenate([jnp.zeros((16, 16), f32), e16], 0))
    xexp = jnp.kron(i8, jnp.zeros((16, 32), f32).at[0].set(1.0))
    fold = jnp.kron(jnp.ones((8, 1), f32), e32)
    expm = jnp.kron(jnp.ones((1, 8), f32), e32)

    def wpack(W):
        return jnp.kron(i8, W[:HH]), jnp.kron(i8, W[HH:])

    w10, w11 = wpack(W1)
    w20, w21 = wpack(W2)
    w30, w31 = wpack(W3)
    b1r = jnp.tile(b1, 8).reshape(1, 256)
    b2r = jnp.tile(b2, 8).reshape(1, 256)
    b3r = jnp.tile(b3, 8).reshape(1, 256)
    g1r, be1r = jnp.tile(g1, 8).reshape(1, 256), jnp.tile(be1, 8).reshape(1, 256)
    g2r, be2r = jnp.tile(g2, 8).reshape(1, 256), jnp.tile(be2, 8).reshape(1, 256)
    gbr = gb.reshape(1, G)
    pW1a, pW1b = pW1[:H], pW1[H:]
    pb1r = pb1.reshape(1, H)
    pW2p = jnp.concatenate([pW2, jnp.zeros((H, 8 - T), f32)], axis=1)
    pb2p = jnp.concatenate([pb2, jnp.zeros((8 - T,), f32)]).reshape(1, 8)

    p = _sc_degree(dst_deg, zeros_nph)
    dqh, dq2, s = _tc_prep(p.reshape(NC, NP8, 128), x_pk, xexp, s0m, s1m)

    def layer(s, w0, w1, br, gr, ber):
        agg = _sc_aggregate(s.reshape(NC, NP, HH), src_p, dst_p)
        y, ssum, ssq = _tc_matstats(agg.reshape(NC, NP8, 128), dqh, w0, w1, br)
        return _tc_bnrelu(y, ssum, ssq, dq2, gr, ber, fold, expm, s0m, s1m)[0]

    s = layer(s, w10, w11, b1r, g1r, be1r)
    s = layer(s, w20, w21, b2r, g2r, be2r)

    agg = _sc_aggregate(s.reshape(NC, NP, HH), src_p, dst_p)
    psum, pcnt = _tc_pool(agg.reshape(NC, NP8, 128), dqh, w30, w31, b3r, bp)

    out = _tc_head(psum, pcnt, global_features, gW, gbr,
                   pW1a, pW1b, pb1r, pW2p, pb2p)
    return out[:, :T]
